# baseline (speedup 1.0000x reference)
import functools

import jax
import jax.numpy as jnp
from jax import lax
from jax.experimental import pallas as pl
from jax.experimental.pallas import tpu as pltpu
from jax.experimental.pallas import tpu_sc as plsc

CHUNK = 128


def _gather_kernel(idx_hbm, table_hbm, out_hbm, idx_v, rows_v, gsem,
                   *, n_chunks):
    wid = lax.axis_index("s") * 2 + lax.axis_index("c")
    pltpu.sync_copy(idx_hbm.at[wid], idx_v)

    def body(g, _):
        copy = pltpu.async_copy(table_hbm.at[idx_v.at[g]], rows_v, gsem)
        copy.wait()
        pltpu.sync_copy(rows_v, out_hbm.at[wid, g])
        return 0

    lax.fori_loop(0, n_chunks, body, 0)


def kernel(x, table):
    b, s = x.shape
    v, d = table.shape
    n = b * s
    nw = 32
    per_w = n // nw
    n_chunks = per_w // CHUNK

    xf = x.reshape(nw, n_chunks, CHUNK).astype(jnp.int32)

    mesh = plsc.VectorSubcoreMesh(core_axis_name="c", subcore_axis_name="s")
    k = functools.partial(
        pl.kernel,
        mesh=mesh,
        compiler_params=pltpu.CompilerParams(use_tc_tiling_on_sc=False),
        out_type=jax.ShapeDtypeStruct((nw, n_chunks, CHUNK, d), jnp.float32),
        scratch_types=[
            pltpu.VMEM((n_chunks, CHUNK), jnp.int32),
            pltpu.VMEM((CHUNK, d), jnp.float32),
            pltpu.SemaphoreType.DMA,
        ],
    )(functools.partial(_gather_kernel, n_chunks=n_chunks))

    out = k(xf, table)
    return out.reshape(b, s, d)


# speedup vs baseline: 1.0889x; 1.0889x over previous
import functools

import jax
import jax.numpy as jnp
from jax import lax
from jax.experimental import pallas as pl
from jax.experimental.pallas import tpu as pltpu
from jax.experimental.pallas import tpu_sc as plsc

CHUNK = 512


def _gather_kernel(idx_hbm, table_hbm, out_hbm, idx_v, rows_v, gsem,
                   *, n_chunks):
    wid = lax.axis_index("s") * 2 + lax.axis_index("c")
    pltpu.sync_copy(idx_hbm.at[wid], idx_v)

    def body(g, _):
        copy = pltpu.async_copy(table_hbm.at[idx_v.at[g]], rows_v, gsem)
        copy.wait()
        pltpu.sync_copy(rows_v, out_hbm.at[wid, g])
        return 0

    lax.fori_loop(0, n_chunks, body, 0)


def kernel(x, table):
    b, s = x.shape
    v, d = table.shape
    n = b * s
    nw = 32
    per_w = n // nw
    n_chunks = per_w // CHUNK

    xf = x.reshape(nw, n_chunks, CHUNK).astype(jnp.int32)

    mesh = plsc.VectorSubcoreMesh(core_axis_name="c", subcore_axis_name="s")
    k = functools.partial(
        pl.kernel,
        mesh=mesh,
        compiler_params=pltpu.CompilerParams(use_tc_tiling_on_sc=False),
        out_type=jax.ShapeDtypeStruct((nw, n_chunks, CHUNK, d), jnp.float32),
        scratch_types=[
            pltpu.VMEM((n_chunks, CHUNK), jnp.int32),
            pltpu.VMEM((CHUNK, d), jnp.float32),
            pltpu.SemaphoreType.DMA,
        ],
    )(functools.partial(_gather_kernel, n_chunks=n_chunks))

    out = k(xf, table)
    return out.reshape(b, s, d)


# speedup vs baseline: 1.1131x; 1.0222x over previous
import functools

import jax
import jax.numpy as jnp
from jax import lax
from jax.experimental import pallas as pl
from jax.experimental.pallas import tpu as pltpu
from jax.experimental.pallas import tpu_sc as plsc

CHUNK = 256
NBUF = 4


def _gather_kernel(idx_hbm, table_hbm, out_hbm, idx_v, bufs, gsems, osems,
                   *, n_chunks):
    wid = lax.axis_index("s") * 2 + lax.axis_index("c")
    pltpu.sync_copy(idx_hbm.at[wid], idx_v)

    def fire_gather(t, k):
        pltpu.async_copy(table_hbm.at[idx_v.at[t]], bufs.at[k], gsems.at[k])

    def wait_gather(k):
        pltpu.make_async_copy(out_hbm.at[wid, 0], bufs.at[k],
                              gsems.at[k]).wait()

    def fire_copy(t, k):
        pltpu.async_copy(bufs.at[k], out_hbm.at[wid, t], osems.at[k])

    def wait_copy(k):
        pltpu.make_async_copy(bufs.at[k], out_hbm.at[wid, 0],
                              osems.at[k]).wait()

    for k in range(NBUF):
        fire_gather(k, k)
        if k >= 1:
            wait_gather(k - 1)
            fire_copy(k - 1, k - 1)

    def body(i, _):
        for k in range(NBUF):
            t = i * NBUF + k
            kp = (k - 1) % NBUF
            wait_copy(k)
            fire_gather(t, k)
            wait_gather(kp)
            fire_copy(t - 1, kp)
        return 0

    lax.fori_loop(1, n_chunks // NBUF, body, 0)

    wait_gather(NBUF - 1)
    fire_copy(n_chunks - 1, NBUF - 1)
    for k in range(NBUF):
        wait_copy(k)


def kernel(x, table):
    b, s = x.shape
    v, d = table.shape
    n = b * s
    nw = 32
    per_w = n // nw
    n_chunks = per_w // CHUNK

    xf = x.reshape(nw, n_chunks, CHUNK).astype(jnp.int32)

    mesh = plsc.VectorSubcoreMesh(core_axis_name="c", subcore_axis_name="s")
    k = functools.partial(
        pl.kernel,
        mesh=mesh,
        compiler_params=pltpu.CompilerParams(use_tc_tiling_on_sc=False),
        out_type=jax.ShapeDtypeStruct((nw, n_chunks, CHUNK, d), jnp.float32),
        scratch_types=[
            pltpu.VMEM((n_chunks, CHUNK), jnp.int32),
            pltpu.VMEM((NBUF, CHUNK, d), jnp.float32),
            pltpu.SemaphoreType.DMA((NBUF,)),
            pltpu.SemaphoreType.DMA((NBUF,)),
        ],
    )(functools.partial(_gather_kernel, n_chunks=n_chunks))

    out = k(xf, table)
    return out.reshape(b, s, d)
